# initial kernel scaffold (unmeasured)
import jax
import jax.numpy as jnp
from jax import lax
from jax.experimental import pallas as pl
from jax.experimental.pallas import tpu as pltpu

N_DEV = 4
M_PER = 1024
K_PER = 1024
N_OUT = 8192


def kernel(x, w_mat, scale_x, scale_w):
    x8 = x.astype(jnp.float8_e4m3fn)
    w8 = w_mat.astype(jnp.float8_e4m3fn)
    s = (scale_x * scale_w).astype(jnp.float32)

    def body(x_ref, w_ref, s_ref, out_ref, xg_ref, wg_ref,
             x_send, x_recv, w_send, w_recv):
        my_i = lax.axis_index("i")

        barrier_sem = pltpu.get_barrier_semaphore()
        for d in range(1, N_DEV):
            peer = lax.rem(my_i + d, N_DEV)
            pl.semaphore_signal(
                barrier_sem, inc=1,
                device_id=(peer,), device_id_type=pl.DeviceIdType.MESH,
            )
        pl.semaphore_wait(barrier_sem, N_DEV - 1)

        rdmas = []
        for d in range(1, N_DEV):
            peer = lax.rem(my_i + d, N_DEV)
            w_rdma = pltpu.make_async_remote_copy(
                src_ref=w_ref,
                dst_ref=wg_ref.at[d - 1],
                send_sem=w_send.at[d - 1],
                recv_sem=w_recv.at[d - 1],
                device_id=(peer,),
                device_id_type=pl.DeviceIdType.MESH,
            )
            w_rdma.start()
            x_rdma = pltpu.make_async_remote_copy(
                src_ref=x_ref.at[pl.ds(peer * M_PER, M_PER), :],
                dst_ref=xg_ref.at[d - 1],
                send_sem=x_send.at[d - 1],
                recv_sem=x_recv.at[d - 1],
                device_id=(peer,),
                device_id_type=pl.DeviceIdType.MESH,
            )
            x_rdma.start()
            rdmas.append((w_rdma, x_rdma))

        xg_ref[N_DEV - 1] = x_ref[pl.ds(my_i * M_PER, M_PER), :]
        wg_ref[N_DEV - 1] = w_ref[:, :]

        out_ref[:, :] = jnp.dot(
            xg_ref[N_DEV - 1], wg_ref[N_DEV - 1],
            preferred_element_type=jnp.float32,
        )

        for d in range(1, N_DEV):
            w_rdma, x_rdma = rdmas[d - 1]
            w_rdma.wait()
            x_rdma.wait()
            out_ref[:, :] += jnp.dot(
                xg_ref[d - 1], wg_ref[d - 1],
                preferred_element_type=jnp.float32,
            )

        out_ref[:, :] = jnp.maximum(out_ref[:, :] * s_ref[0], 0.0)

    return pl.pallas_call(
        body,
        out_shape=jax.ShapeDtypeStruct((M_PER, N_OUT), jnp.float32),
        in_specs=[
            pl.BlockSpec(memory_space=pltpu.VMEM),
            pl.BlockSpec(memory_space=pltpu.VMEM),
            pl.BlockSpec(memory_space=pltpu.SMEM),
        ],
        out_specs=pl.BlockSpec(memory_space=pltpu.VMEM),
        scratch_shapes=[
            pltpu.VMEM((N_DEV, M_PER, K_PER), jnp.float8_e4m3fn),
            pltpu.VMEM((N_DEV, K_PER, N_OUT), jnp.float8_e4m3fn),
            pltpu.SemaphoreType.DMA((N_DEV - 1,)),
            pltpu.SemaphoreType.DMA((N_DEV - 1,)),
            pltpu.SemaphoreType.DMA((N_DEV - 1,)),
            pltpu.SemaphoreType.DMA((N_DEV - 1,)),
        ],
        compiler_params=pltpu.CompilerParams(collective_id=0),
    )(x8, w8, s)


# baseline (device time: 309158 ns/iter reference)
import jax
import jax.numpy as jnp
from jax import lax
from jax.experimental import pallas as pl
from jax.experimental.pallas import tpu as pltpu

N_DEV = 4
M_PER = 1024
K_PER = 1024
N_OUT = 8192
N_CHUNK = 2048


def kernel(x, w_mat, scale_x, scale_w):
    x8 = x.astype(jnp.float8_e4m3fn)
    w8 = w_mat.astype(jnp.float8_e4m3fn)
    s = (scale_x * scale_w).astype(jnp.float32)

    def body(x_ref, w_ref, s_ref, out_ref, xg_ref, wg_ref, acc_ref,
             x_send, x_recv, w_send, w_recv, copy_sem):
        my_i = lax.axis_index("i")

        barrier_sem = pltpu.get_barrier_semaphore()
        for d in range(1, N_DEV):
            peer = lax.rem(my_i + d, N_DEV)
            pl.semaphore_signal(
                barrier_sem, inc=1,
                device_id=(peer,), device_id_type=pl.DeviceIdType.MESH,
            )
        pl.semaphore_wait(barrier_sem, N_DEV - 1)

        rdmas = []
        for d in range(1, N_DEV):
            peer = lax.rem(my_i + d, N_DEV)
            w_rdma = pltpu.make_async_remote_copy(
                src_ref=w_ref,
                dst_ref=wg_ref.at[d - 1],
                send_sem=w_send.at[d - 1],
                recv_sem=w_recv.at[d - 1],
                device_id=(peer,),
                device_id_type=pl.DeviceIdType.MESH,
            )
            w_rdma.start()
            x_rdma = pltpu.make_async_remote_copy(
                src_ref=x_ref.at[pl.ds(peer * M_PER, M_PER), :],
                dst_ref=xg_ref.at[d - 1],
                send_sem=x_send.at[d - 1],
                recv_sem=x_recv.at[d - 1],
                device_id=(peer,),
                device_id_type=pl.DeviceIdType.MESH,
            )
            x_rdma.start()
            rdmas.append((w_rdma, x_rdma))

        for w_rdma, x_rdma in rdmas:
            w_rdma.wait()
            x_rdma.wait()

        x_loc = x_ref[pl.ds(my_i * M_PER, M_PER), :]
        for c in range(N_OUT // N_CHUNK):
            col = pl.ds(c * N_CHUNK, N_CHUNK)
            acc_ref[:, :] = jnp.dot(
                x_loc, w_ref[:, col], preferred_element_type=jnp.float32,
            )
            for d in range(1, N_DEV):
                acc_ref[:, :] += jnp.dot(
                    xg_ref[d - 1], wg_ref[d - 1, :, col],
                    preferred_element_type=jnp.float32,
                )
            acc_ref[:, :] = jnp.maximum(acc_ref[:, :] * s_ref[0], 0.0)
            out_copy = pltpu.make_async_copy(
                acc_ref, out_ref.at[:, col], copy_sem,
            )
            out_copy.start()
            out_copy.wait()

    return pl.pallas_call(
        body,
        out_shape=jax.ShapeDtypeStruct((M_PER, N_OUT), jnp.float32),
        in_specs=[
            pl.BlockSpec(memory_space=pltpu.VMEM),
            pl.BlockSpec(memory_space=pltpu.VMEM),
            pl.BlockSpec(memory_space=pltpu.SMEM),
        ],
        out_specs=pl.BlockSpec(memory_space=pl.ANY),
        scratch_shapes=[
            pltpu.VMEM((N_DEV - 1, M_PER, K_PER), jnp.float8_e4m3fn),
            pltpu.VMEM((N_DEV - 1, K_PER, N_OUT), jnp.float8_e4m3fn),
            pltpu.VMEM((M_PER, N_CHUNK), jnp.float32),
            pltpu.SemaphoreType.DMA((N_DEV - 1,)),
            pltpu.SemaphoreType.DMA((N_DEV - 1,)),
            pltpu.SemaphoreType.DMA((N_DEV - 1,)),
            pltpu.SemaphoreType.DMA((N_DEV - 1,)),
            pltpu.SemaphoreType.DMA,
        ],
        compiler_params=pltpu.CompilerParams(
            collective_id=0,
            vmem_limit_bytes=100 * 1024 * 1024,
        ),
    )(x8, w8, s)


# device time: 290120 ns/iter; 1.0656x vs baseline; 1.0656x over previous
import jax
import jax.numpy as jnp
from jax import lax
from jax.experimental import pallas as pl
from jax.experimental.pallas import tpu as pltpu

N_DEV = 4
M_PER = 1024
K_PER = 1024
N_OUT = 8192
N_STAGE = 512


def kernel(x, w_mat, scale_x, scale_w):
    x8 = x.astype(jnp.float8_e4m3fn)
    w8 = w_mat.astype(jnp.float8_e4m3fn)
    s = (scale_x * scale_w).astype(jnp.float32)

    def body(x_ref, w_ref, s_ref, out_ref, xg_ref, wg_ref, acc_ref,
             stage_ref, x_send, x_recv, w_send, w_recv, copy_sems):
        my_i = lax.axis_index("i")

        barrier_sem = pltpu.get_barrier_semaphore()
        for d in range(1, N_DEV):
            peer = lax.rem(my_i + d, N_DEV)
            pl.semaphore_signal(
                barrier_sem, inc=1,
                device_id=(peer,), device_id_type=pl.DeviceIdType.MESH,
            )
        pl.semaphore_wait(barrier_sem, N_DEV - 1)

        rdmas = []
        for d in range(1, N_DEV):
            peer = lax.rem(my_i + d, N_DEV)
            w_rdma = pltpu.make_async_remote_copy(
                src_ref=w_ref,
                dst_ref=wg_ref.at[d - 1],
                send_sem=w_send.at[d - 1],
                recv_sem=w_recv.at[d - 1],
                device_id=(peer,),
                device_id_type=pl.DeviceIdType.MESH,
            )
            w_rdma.start()
            x_rdma = pltpu.make_async_remote_copy(
                src_ref=x_ref.at[pl.ds(peer * M_PER, M_PER), :],
                dst_ref=xg_ref.at[d - 1],
                send_sem=x_send.at[d - 1],
                recv_sem=x_recv.at[d - 1],
                device_id=(peer,),
                device_id_type=pl.DeviceIdType.MESH,
            )
            x_rdma.start()
            rdmas.append((w_rdma, x_rdma))

        acc_ref[:, :] = jnp.dot(
            x_ref[pl.ds(my_i * M_PER, M_PER), :], w_ref[:, :],
            preferred_element_type=jnp.float32,
        ).astype(jnp.bfloat16)

        for d in (1, 3, 2):
            w_rdma, x_rdma = rdmas[d - 1]
            x_rdma.wait()
            w_rdma.wait()
            acc_ref[:, :] = (
                acc_ref[:, :].astype(jnp.float32)
                + jnp.dot(
                    xg_ref[d - 1], wg_ref[d - 1],
                    preferred_element_type=jnp.float32,
                )
            ).astype(jnp.bfloat16)

        copies = [None, None]
        for c in range(N_OUT // N_STAGE):
            slot = c % 2
            if copies[slot] is not None:
                copies[slot].wait()
            col = pl.ds(c * N_STAGE, N_STAGE)
            stage_ref[slot] = jnp.maximum(
                acc_ref[:, col].astype(jnp.float32) * s_ref[0], 0.0
            )
            cp = pltpu.make_async_copy(
                stage_ref.at[slot], out_ref.at[:, col], copy_sems.at[slot],
            )
            cp.start()
            copies[slot] = cp
        for cp in copies:
            cp.wait()

    return pl.pallas_call(
        body,
        out_shape=jax.ShapeDtypeStruct((M_PER, N_OUT), jnp.float32),
        in_specs=[
            pl.BlockSpec(memory_space=pltpu.VMEM),
            pl.BlockSpec(memory_space=pltpu.VMEM),
            pl.BlockSpec(memory_space=pltpu.SMEM),
        ],
        out_specs=pl.BlockSpec(memory_space=pl.ANY),
        scratch_shapes=[
            pltpu.VMEM((N_DEV - 1, M_PER, K_PER), jnp.float8_e4m3fn),
            pltpu.VMEM((N_DEV - 1, K_PER, N_OUT), jnp.float8_e4m3fn),
            pltpu.VMEM((M_PER, N_OUT), jnp.bfloat16),
            pltpu.VMEM((2, M_PER, N_STAGE), jnp.float32),
            pltpu.SemaphoreType.DMA((N_DEV - 1,)),
            pltpu.SemaphoreType.DMA((N_DEV - 1,)),
            pltpu.SemaphoreType.DMA((N_DEV - 1,)),
            pltpu.SemaphoreType.DMA((N_DEV - 1,)),
            pltpu.SemaphoreType.DMA((2,)),
        ],
        compiler_params=pltpu.CompilerParams(
            collective_id=0,
            vmem_limit_bytes=100 * 1024 * 1024,
        ),
    )(x8, w8, s)


# device time: 272983 ns/iter; 1.1325x vs baseline; 1.0628x over previous
import jax
import jax.numpy as jnp
from jax import lax
from jax.experimental import pallas as pl
from jax.experimental.pallas import tpu as pltpu

N_DEV = 4
M_PER = 1024
K_PER = 1024
N_OUT = 8192
N_STAGE = 512
W_CHUNK = 128
N_WCHUNKS = K_PER // W_CHUNK


def kernel(x, w_mat, scale_x, scale_w):
    x8 = x.astype(jnp.float8_e4m3fn)
    s = (scale_x * scale_w).astype(jnp.float32)

    def body(x_ref, w_hbm, s_ref, out_ref, xg_ref, wg_ref, w8_ref,
             wstage_ref, acc_ref, stage_ref,
             x_send, x_recv, w_send, w_recv, w_in_sem, copy_sems):
        my_i = lax.axis_index("i")

        barrier_sem = pltpu.get_barrier_semaphore()
        for d in range(1, N_DEV):
            peer = lax.rem(my_i + d, N_DEV)
            pl.semaphore_signal(
                barrier_sem, inc=1,
                device_id=(peer,), device_id_type=pl.DeviceIdType.MESH,
            )
        pl.semaphore_wait(barrier_sem, N_DEV - 1)

        x_rdmas = {}
        for d in range(1, N_DEV):
            peer = lax.rem(my_i + d, N_DEV)
            x_rdmas[d] = pltpu.make_async_remote_copy(
                src_ref=x_ref.at[pl.ds(peer * M_PER, M_PER), :],
                dst_ref=xg_ref.at[d - 1],
                send_sem=x_send.at[d - 1],
                recv_sem=x_recv.at[d - 1],
                device_id=(peer,),
                device_id_type=pl.DeviceIdType.MESH,
            )
            x_rdmas[d].start()

        w_rdmas = {}
        for k in range(N_WCHUNKS):
            rows = pl.ds(k * W_CHUNK, W_CHUNK)
            w_in = pltpu.make_async_copy(
                w_hbm.at[rows, :], wstage_ref, w_in_sem,
            )
            w_in.start()
            w_in.wait()
            w8_ref[rows, :] = wstage_ref[:, :].astype(jnp.float8_e4m3fn)
            for d in range(1, N_DEV):
                peer = lax.rem(my_i + d, N_DEV)
                r = pltpu.make_async_remote_copy(
                    src_ref=w8_ref.at[rows, :],
                    dst_ref=wg_ref.at[d - 1, rows, :],
                    send_sem=w_send.at[d - 1, k],
                    recv_sem=w_recv.at[d - 1, k],
                    device_id=(peer,),
                    device_id_type=pl.DeviceIdType.MESH,
                )
                r.start()
                w_rdmas[d, k] = r

        acc_ref[:, :] = jnp.dot(
            x_ref[pl.ds(my_i * M_PER, M_PER), :], w8_ref[:, :],
            preferred_element_type=jnp.float32,
        ).astype(jnp.bfloat16)

        for d in (1, 3, 2):
            x_rdmas[d].wait()
            for k in range(N_WCHUNKS):
                w_rdmas[d, k].wait()
            acc_ref[:, :] = (
                acc_ref[:, :].astype(jnp.float32)
                + jnp.dot(
                    xg_ref[d - 1], wg_ref[d - 1],
                    preferred_element_type=jnp.float32,
                )
            ).astype(jnp.bfloat16)

        copies = [None, None]
        for c in range(N_OUT // N_STAGE):
            slot = c % 2
            if copies[slot] is not None:
                copies[slot].wait()
            col = pl.ds(c * N_STAGE, N_STAGE)
            stage_ref[slot] = jnp.maximum(
                acc_ref[:, col].astype(jnp.float32) * s_ref[0], 0.0
            )
            cp = pltpu.make_async_copy(
                stage_ref.at[slot], out_ref.at[:, col], copy_sems.at[slot],
            )
            cp.start()
            copies[slot] = cp
        for cp in copies:
            cp.wait()

    return pl.pallas_call(
        body,
        out_shape=jax.ShapeDtypeStruct((M_PER, N_OUT), jnp.float32),
        in_specs=[
            pl.BlockSpec(memory_space=pltpu.MemorySpace.VMEM),
            pl.BlockSpec(memory_space=pl.ANY),
            pl.BlockSpec(memory_space=pltpu.MemorySpace.SMEM),
        ],
        out_specs=pl.BlockSpec(memory_space=pl.ANY),
        scratch_shapes=[
            pltpu.VMEM((N_DEV - 1, M_PER, K_PER), jnp.float8_e4m3fn),
            pltpu.VMEM((N_DEV - 1, K_PER, N_OUT), jnp.float8_e4m3fn),
            pltpu.VMEM((K_PER, N_OUT), jnp.float8_e4m3fn),
            pltpu.VMEM((W_CHUNK, N_OUT), jnp.float32),
            pltpu.VMEM((M_PER, N_OUT), jnp.bfloat16),
            pltpu.VMEM((2, M_PER, N_STAGE), jnp.float32),
            pltpu.SemaphoreType.DMA((N_DEV - 1,)),
            pltpu.SemaphoreType.DMA((N_DEV - 1,)),
            pltpu.SemaphoreType.DMA((N_DEV - 1, N_WCHUNKS)),
            pltpu.SemaphoreType.DMA((N_DEV - 1, N_WCHUNKS)),
            pltpu.SemaphoreType.DMA,
            pltpu.SemaphoreType.DMA((2,)),
        ],
        compiler_params=pltpu.CompilerParams(
            collective_id=0,
            vmem_limit_bytes=100 * 1024 * 1024,
        ),
    )(x8, w_mat, s)


# device time: 270880 ns/iter; 1.1413x vs baseline; 1.0078x over previous
import jax
import jax.numpy as jnp
from jax import lax
from jax.experimental import pallas as pl
from jax.experimental.pallas import tpu as pltpu

N_DEV = 4
M_PER = 1024
K_PER = 1024
N_OUT = 8192
N_STAGE = 256
X_CHUNK = 512
W_CHUNK = 128
N_WCHUNKS = K_PER // W_CHUNK


def kernel(x, w_mat, scale_x, scale_w):
    s = (scale_x * scale_w).astype(jnp.float32)

    def body(x_hbm, w_hbm, s_ref, out_ref, xg_ref, wg_ref, w8_ref,
             wstage_ref, x8_ref, xstage_ref, acc_ref, stage_ref,
             x_send, x_recv, w_send, w_recv, w_in_sem, x_in_sem,
             copy_sems):
        my_i = lax.axis_index("i")

        barrier_sem = pltpu.get_barrier_semaphore()
        for d in range(1, N_DEV):
            peer = lax.rem(my_i + d, N_DEV)
            pl.semaphore_signal(
                barrier_sem, inc=1,
                device_id=(peer,), device_id_type=pl.DeviceIdType.MESH,
            )
        pl.semaphore_wait(barrier_sem, N_DEV - 1)

        w_rdmas = {}
        for k in range(N_WCHUNKS):
            rows = pl.ds(k * W_CHUNK, W_CHUNK)
            w_in = pltpu.make_async_copy(
                w_hbm.at[rows, :], wstage_ref, w_in_sem,
            )
            w_in.start()
            w_in.wait()
            w8_ref[rows, :] = wstage_ref[:, :].astype(jnp.float8_e4m3fn)
            for d in range(1, N_DEV):
                peer = lax.rem(my_i + d, N_DEV)
                r = pltpu.make_async_remote_copy(
                    src_ref=w8_ref.at[rows, :],
                    dst_ref=wg_ref.at[d - 1, rows, :],
                    send_sem=w_send.at[d - 1, k],
                    recv_sem=w_recv.at[d - 1, k],
                    device_id=(peer,),
                    device_id_type=pl.DeviceIdType.MESH,
                )
                r.start()
                w_rdmas[d, k] = r

        def convert_x_block(base):
            for h in range(M_PER // X_CHUNK):
                rows = pl.ds(base + h * X_CHUNK, X_CHUNK)
                x_in = pltpu.make_async_copy(
                    x_hbm.at[rows, :], xstage_ref, x_in_sem,
                )
                x_in.start()
                x_in.wait()
                x8_ref[rows, :] = xstage_ref[:, :].astype(jnp.float8_e4m3fn)

        x_rdmas = {}
        for d in range(1, N_DEV):
            peer = lax.rem(my_i + d, N_DEV)
            convert_x_block(peer * M_PER)
            x_rdmas[d] = pltpu.make_async_remote_copy(
                src_ref=x8_ref.at[pl.ds(peer * M_PER, M_PER), :],
                dst_ref=xg_ref.at[d - 1],
                send_sem=x_send.at[d - 1],
                recv_sem=x_recv.at[d - 1],
                device_id=(peer,),
                device_id_type=pl.DeviceIdType.MESH,
            )
            x_rdmas[d].start()
        convert_x_block(my_i * M_PER)

        acc_ref[:, :] = jnp.dot(
            x8_ref[pl.ds(my_i * M_PER, M_PER), :], w8_ref[:, :],
            preferred_element_type=jnp.float32,
        ).astype(jnp.bfloat16)

        for d in (1, 3, 2):
            x_rdmas[d].wait()
            for k in range(N_WCHUNKS):
                w_rdmas[d, k].wait()
            acc_ref[:, :] = (
                acc_ref[:, :].astype(jnp.float32)
                + jnp.dot(
                    xg_ref[d - 1], wg_ref[d - 1],
                    preferred_element_type=jnp.float32,
                )
            ).astype(jnp.bfloat16)

        copies = [None, None]
        for c in range(N_OUT // N_STAGE):
            slot = c % 2
            if copies[slot] is not None:
                copies[slot].wait()
            col = pl.ds(c * N_STAGE, N_STAGE)
            stage_ref[slot] = jnp.maximum(
                acc_ref[:, col].astype(jnp.float32) * s_ref[0], 0.0
            )
            cp = pltpu.make_async_copy(
                stage_ref.at[slot], out_ref.at[:, col], copy_sems.at[slot],
            )
            cp.start()
            copies[slot] = cp
        for cp in copies:
            cp.wait()

    return pl.pallas_call(
        body,
        out_shape=jax.ShapeDtypeStruct((M_PER, N_OUT), jnp.float32),
        in_specs=[
            pl.BlockSpec(memory_space=pl.ANY),
            pl.BlockSpec(memory_space=pl.ANY),
            pl.BlockSpec(memory_space=pltpu.MemorySpace.SMEM),
        ],
        out_specs=pl.BlockSpec(memory_space=pl.ANY),
        scratch_shapes=[
            pltpu.VMEM((N_DEV - 1, M_PER, K_PER), jnp.float8_e4m3fn),
            pltpu.VMEM((N_DEV - 1, K_PER, N_OUT), jnp.float8_e4m3fn),
            pltpu.VMEM((K_PER, N_OUT), jnp.float8_e4m3fn),
            pltpu.VMEM((W_CHUNK, N_OUT), jnp.float32),
            pltpu.VMEM((N_DEV * M_PER, K_PER), jnp.float8_e4m3fn),
            pltpu.VMEM((X_CHUNK, K_PER), jnp.float32),
            pltpu.VMEM((M_PER, N_OUT), jnp.bfloat16),
            pltpu.VMEM((2, M_PER, N_STAGE), jnp.float32),
            pltpu.SemaphoreType.DMA((N_DEV - 1,)),
            pltpu.SemaphoreType.DMA((N_DEV - 1,)),
            pltpu.SemaphoreType.DMA((N_DEV - 1, N_WCHUNKS)),
            pltpu.SemaphoreType.DMA((N_DEV - 1, N_WCHUNKS)),
            pltpu.SemaphoreType.DMA,
            pltpu.SemaphoreType.DMA,
            pltpu.SemaphoreType.DMA((2,)),
        ],
        compiler_params=pltpu.CompilerParams(
            collective_id=0,
            vmem_limit_bytes=100 * 1024 * 1024,
        ),
    )(x, w_mat, s)


# device time: 268798 ns/iter; 1.1501x vs baseline; 1.0077x over previous
import jax
import jax.numpy as jnp
from jax import lax
from jax.experimental import pallas as pl
from jax.experimental.pallas import tpu as pltpu

N_DEV = 4
M_PER = 1024
K_PER = 1024
N_OUT = 8192
N_STAGE = 256
X_CHUNK = 512
W_CHUNK = 128
N_WCHUNKS = K_PER // W_CHUNK


def kernel(x, w_mat, scale_x, scale_w):
    s = (scale_x * scale_w).astype(jnp.float32)

    def body(x_hbm, w_hbm, s_ref, out_ref, xg_ref, wg_ref, w8_ref,
             wstage_ref, x8_ref, xstage_ref, acc_ref, stage_ref,
             x_send, x_recv, w_send, w_recv, w_in_sem, x_in_sem,
             copy_sems):
        my_i = lax.axis_index("i")

        w_in0 = pltpu.make_async_copy(
            w_hbm.at[pl.ds(0, W_CHUNK), :], wstage_ref, w_in_sem,
        )
        w_in0.start()

        barrier_sem = pltpu.get_barrier_semaphore()
        for d in range(1, N_DEV):
            peer = lax.rem(my_i + d, N_DEV)
            pl.semaphore_signal(
                barrier_sem, inc=1,
                device_id=(peer,), device_id_type=pl.DeviceIdType.MESH,
            )
        pl.semaphore_wait(barrier_sem, N_DEV - 1)

        w_rdmas = {}
        for k in range(N_WCHUNKS):
            rows = pl.ds(k * W_CHUNK, W_CHUNK)
            w_in = pltpu.make_async_copy(
                w_hbm.at[rows, :], wstage_ref, w_in_sem,
            )
            if k > 0:
                w_in.start()
            w_in.wait()
            w8_ref[rows, :] = wstage_ref[:, :].astype(jnp.float8_e4m3fn)
            for d in range(1, N_DEV):
                peer = lax.rem(my_i + d, N_DEV)
                r = pltpu.make_async_remote_copy(
                    src_ref=w8_ref.at[rows, :],
                    dst_ref=wg_ref.at[d - 1, rows, :],
                    send_sem=w_send.at[d - 1, k],
                    recv_sem=w_recv.at[d - 1, k],
                    device_id=(peer,),
                    device_id_type=pl.DeviceIdType.MESH,
                )
                r.start()
                w_rdmas[d, k] = r

        def convert_x_block(base):
            for h in range(M_PER // X_CHUNK):
                rows = pl.ds(base + h * X_CHUNK, X_CHUNK)
                x_in = pltpu.make_async_copy(
                    x_hbm.at[rows, :], xstage_ref, x_in_sem,
                )
                x_in.start()
                x_in.wait()
                x8_ref[rows, :] = xstage_ref[:, :].astype(jnp.float8_e4m3fn)

        x_rdmas = {}
        for d in range(1, N_DEV):
            peer = lax.rem(my_i + d, N_DEV)
            convert_x_block(peer * M_PER)
            x_rdmas[d] = pltpu.make_async_remote_copy(
                src_ref=x8_ref.at[pl.ds(peer * M_PER, M_PER), :],
                dst_ref=xg_ref.at[d - 1],
                send_sem=x_send.at[d - 1],
                recv_sem=x_recv.at[d - 1],
                device_id=(peer,),
                device_id_type=pl.DeviceIdType.MESH,
            )
            x_rdmas[d].start()
        convert_x_block(my_i * M_PER)

        acc_ref[:, :] = jnp.dot(
            x8_ref[pl.ds(my_i * M_PER, M_PER), :], w8_ref[:, :],
            preferred_element_type=jnp.float32,
        ).astype(jnp.bfloat16)

        for d in (1, 3):
            x_rdmas[d].wait()
            for k in range(N_WCHUNKS):
                w_rdmas[d, k].wait()
            acc_ref[:, :] = (
                acc_ref[:, :].astype(jnp.float32)
                + jnp.dot(
                    xg_ref[d - 1], wg_ref[d - 1],
                    preferred_element_type=jnp.float32,
                )
            ).astype(jnp.bfloat16)

        x_rdmas[2].wait()
        for k in range(N_WCHUNKS):
            w_rdmas[2, k].wait()

        copies = [None, None]
        for c in range(N_OUT // N_STAGE):
            slot = c % 2
            if copies[slot] is not None:
                copies[slot].wait()
            col = pl.ds(c * N_STAGE, N_STAGE)
            stage_ref[slot] = jnp.maximum(
                (
                    acc_ref[:, col].astype(jnp.float32)
                    + jnp.dot(
                        xg_ref[1], wg_ref[1, :, col],
                        preferred_element_type=jnp.float32,
                    )
                )
                * s_ref[0],
                0.0,
            )
            cp = pltpu.make_async_copy(
                stage_ref.at[slot], out_ref.at[:, col], copy_sems.at[slot],
            )
            cp.start()
            copies[slot] = cp
        for cp in copies:
            cp.wait()

    return pl.pallas_call(
        body,
        out_shape=jax.ShapeDtypeStruct((M_PER, N_OUT), jnp.float32),
        in_specs=[
            pl.BlockSpec(memory_space=pl.ANY),
            pl.BlockSpec(memory_space=pl.ANY),
            pl.BlockSpec(memory_space=pltpu.MemorySpace.SMEM),
        ],
        out_specs=pl.BlockSpec(memory_space=pl.ANY),
        scratch_shapes=[
            pltpu.VMEM((N_DEV - 1, M_PER, K_PER), jnp.float8_e4m3fn),
            pltpu.VMEM((N_DEV - 1, K_PER, N_OUT), jnp.float8_e4m3fn),
            pltpu.VMEM((K_PER, N_OUT), jnp.float8_e4m3fn),
            pltpu.VMEM((W_CHUNK, N_OUT), jnp.float32),
            pltpu.VMEM((N_DEV * M_PER, K_PER), jnp.float8_e4m3fn),
            pltpu.VMEM((X_CHUNK, K_PER), jnp.float32),
            pltpu.VMEM((M_PER, N_OUT), jnp.bfloat16),
            pltpu.VMEM((2, M_PER, N_STAGE), jnp.float32),
            pltpu.SemaphoreType.DMA((N_DEV - 1,)),
            pltpu.SemaphoreType.DMA((N_DEV - 1,)),
            pltpu.SemaphoreType.DMA((N_DEV - 1, N_WCHUNKS)),
            pltpu.SemaphoreType.DMA((N_DEV - 1, N_WCHUNKS)),
            pltpu.SemaphoreType.DMA,
            pltpu.SemaphoreType.DMA,
            pltpu.SemaphoreType.DMA((2,)),
        ],
        compiler_params=pltpu.CompilerParams(
            collective_id=0,
            vmem_limit_bytes=100 * 1024 * 1024,
        ),
    )(x, w_mat, s)
